# Initial kernel scaffold; baseline (speedup 1.0000x reference)
import jax
import jax.numpy as jnp
from jax.experimental import pallas as pl


def kernel(x, edge_index, edge_attr, We_0, be_0, W1_0, b1_0, W2_0, b2_0, gamma_0, beta_0, We_1, be_1, W1_1, b1_1, W2_1, b2_1, gamma_1, beta_1):
    raise NotImplementedError("write your pallas kernel here")


# speedup vs baseline: 1.8298x; 1.8298x over previous
import functools

import jax
import jax.numpy as jnp
from jax import lax
from jax.experimental import pallas as pl
from jax.experimental.pallas import tpu as pltpu
from jax.experimental.pallas import tpu_sc as plsc

N_NODES = 10000
N_EDGES = 320000
FDIM = 128
EDIM = 16

NC = 2
NS = 16
CHUNK = 128
E_PAD = 327680
EPW = E_PAD // (NC * NS)
NCHUNKS = EPW // CHUNK
N_PAD = 10240
RPT = N_PAD // NS


def _edge_body(ea_ref, we_ref, be_ref, out_ref):
    out_ref[...] = (
        jnp.dot(ea_ref[...], we_ref[...], preferred_element_type=jnp.float32)
        + be_ref[...]
    )


def _tc_edge_embed(ea, We, be):
    BE = 4096
    return pl.pallas_call(
        _edge_body,
        grid=(E_PAD // BE,),
        in_specs=[
            pl.BlockSpec((BE, EDIM), lambda i: (i, 0)),
            pl.BlockSpec((EDIM, FDIM), lambda i: (0, 0)),
            pl.BlockSpec((1, FDIM), lambda i: (0, 0)),
        ],
        out_specs=pl.BlockSpec((BE, FDIM), lambda i: (i, 0)),
        out_shape=jax.ShapeDtypeStruct((E_PAD, FDIM), jnp.float32),
    )(ea, We, be)


def _mlp_body(h_ref, a0_ref, a1_ref, w1_ref, b1_ref, w2_ref, b2_ref,
              g_ref, bb_ref, o_ref):
    z = h_ref[...] + a0_ref[...] + a1_ref[...]
    t = jnp.dot(z, w1_ref[...], preferred_element_type=jnp.float32) + b1_ref[...]
    t = jnp.maximum(t, 0.0)
    t = jnp.dot(t, w2_ref[...], preferred_element_type=jnp.float32) + b2_ref[...]
    mu = jnp.mean(t, axis=0, keepdims=True)
    d = t - mu
    var = jnp.mean(d * d, axis=0, keepdims=True)
    o_ref[...] = jnp.maximum(
        d * lax.rsqrt(var + 1e-5) * g_ref[...] + bb_ref[...], 0.0)


def _tc_mlp(h, a0, a1, W1, b1, W2, b2, gamma, beta):
    return pl.pallas_call(
        _mlp_body,
        out_shape=jax.ShapeDtypeStruct((N_NODES, FDIM), jnp.float32),
    )(h, a0, a1, W1, b1, W2, b2, gamma, beta)


def _sc_message_pass(h, src, dst, e):
    mesh = plsc.VectorSubcoreMesh(core_axis_name="c", subcore_axis_name="s")

    @functools.partial(
        pl.kernel,
        mesh=mesh,
        out_type=jax.ShapeDtypeStruct((NC, N_PAD, FDIM), jnp.float32),
        scratch_types=[
            pltpu.VMEM((CHUNK,), jnp.int32),
            pltpu.VMEM((CHUNK,), jnp.int32),
            pltpu.VMEM((CHUNK, FDIM), jnp.float32),
            pltpu.VMEM((CHUNK, FDIM), jnp.float32),
            pltpu.VMEM_SHARED((N_PAD, FDIM), jnp.float32),
            pltpu.SemaphoreType.DMA,
        ],
    )
    def mp(h_hbm, src_hbm, dst_hbm, e_hbm, out_hbm,
           sidx, didx, ebuf, hbuf, agg, sem):
        c = lax.axis_index("c")
        s = lax.axis_index("s")

        def zrow(r, carry):
            zv = jnp.zeros((16,), jnp.float32)
            for j in range(FDIM // 16):
                hbuf[r, pl.ds(j * 16, 16)] = zv
            return carry
        lax.fori_loop(0, CHUNK, zrow, 0)
        for k in range(RPT // CHUNK):
            pltpu.sync_copy(hbuf, agg.at[pl.ds(s * RPT + k * CHUNK, CHUNK)])
        plsc.subcore_barrier()

        w = c * NS + s

        def body(k, carry):
            off = w * EPW + k * CHUNK
            pltpu.sync_copy(src_hbm.at[pl.ds(off, CHUNK)], sidx)
            pltpu.sync_copy(dst_hbm.at[pl.ds(off, CHUNK)], didx)
            pltpu.sync_copy(e_hbm.at[pl.ds(off, CHUNK)], ebuf)
            pltpu.async_copy(h_hbm.at[sidx], hbuf, sem).wait()

            def row(r, rc):
                for j in range(FDIM // 16):
                    sl = pl.ds(j * 16, 16)
                    ebuf[r, sl] = jnp.maximum(ebuf[r, sl] + hbuf[r, sl], 0.0)
                return rc
            lax.fori_loop(0, CHUNK, row, 0)

            pltpu.sync_copy(ebuf, agg.at[didx], add=True)
            return carry
        lax.fori_loop(0, NCHUNKS, body, 0)

        plsc.subcore_barrier()
        for k in range(RPT // CHUNK):
            r0 = s * RPT + k * CHUNK
            pltpu.sync_copy(agg.at[pl.ds(r0, CHUNK)], out_hbm.at[c, pl.ds(r0, CHUNK)])

    return mp(h, src, dst, e)


def kernel(x, edge_index, edge_attr,
           We_0, be_0, W1_0, b1_0, W2_0, b2_0, gamma_0, beta_0,
           We_1, be_1, W1_1, b1_1, W2_1, b2_1, gamma_1, beta_1):
    pad = E_PAD - N_EDGES
    src_p = jnp.concatenate([edge_index[0], jnp.zeros((pad,), jnp.int32)])
    dst_p = jnp.concatenate([edge_index[1], jnp.full((pad,), N_NODES, jnp.int32)])
    ea_p = jnp.concatenate([edge_attr, jnp.zeros((pad, EDIM), jnp.float32)])

    h = x
    for (We, be, W1, b1, W2, b2, gamma, beta) in (
        (We_0, be_0, W1_0, b1_0, W2_0, b2_0, gamma_0, beta_0),
        (We_1, be_1, W1_1, b1_1, W2_1, b2_1, gamma_1, beta_1),
    ):
        e = _tc_edge_embed(ea_p, We, be.reshape(1, FDIM))
        agg = _sc_message_pass(h, src_p, dst_p, e)
        h = _tc_mlp(h, agg[0, :N_NODES], agg[1, :N_NODES],
                    W1, b1.reshape(1, FDIM), W2, b2.reshape(1, FDIM),
                    gamma.reshape(1, FDIM), beta.reshape(1, FDIM))
    return h


# speedup vs baseline: 2.7655x; 1.5114x over previous
import functools

import jax
import jax.numpy as jnp
from jax import lax
from jax.experimental import pallas as pl
from jax.experimental.pallas import tpu as pltpu
from jax.experimental.pallas import tpu_sc as plsc

N_NODES = 10000
N_EDGES = 320000
FDIM = 128
EDIM = 16

NC = 2
NS = 16
CHUNK = 128
E_PAD = 327680
EPW = E_PAD // (NC * NS)
NCHUNKS = EPW // CHUNK
N_PAD = 10240
RPT = N_PAD // NS


def _edge_body(ea_ref, we_ref, be_ref, out_ref):
    out_ref[...] = (
        jnp.dot(ea_ref[...], we_ref[...], preferred_element_type=jnp.float32)
        + be_ref[...]
    )


def _tc_edge_embed(ea, We, be):
    BE = 4096
    return pl.pallas_call(
        _edge_body,
        grid=(E_PAD // BE,),
        in_specs=[
            pl.BlockSpec((BE, EDIM), lambda i: (i, 0)),
            pl.BlockSpec((EDIM, FDIM), lambda i: (0, 0)),
            pl.BlockSpec((1, FDIM), lambda i: (0, 0)),
        ],
        out_specs=pl.BlockSpec((BE, FDIM), lambda i: (i, 0)),
        out_shape=jax.ShapeDtypeStruct((E_PAD, FDIM), jnp.float32),
    )(ea, We, be)


def _mlp_body(h_ref, a0_ref, a1_ref, w1_ref, b1_ref, w2_ref, b2_ref,
              g_ref, bb_ref, o_ref):
    z = h_ref[...] + a0_ref[...] + a1_ref[...]
    t = jnp.dot(z, w1_ref[...], preferred_element_type=jnp.float32) + b1_ref[...]
    t = jnp.maximum(t, 0.0)
    t = jnp.dot(t, w2_ref[...], preferred_element_type=jnp.float32) + b2_ref[...]
    mu = jnp.mean(t, axis=0, keepdims=True)
    d = t - mu
    var = jnp.mean(d * d, axis=0, keepdims=True)
    o_ref[...] = jnp.maximum(
        d * lax.rsqrt(var + 1e-5) * g_ref[...] + bb_ref[...], 0.0)


def _tc_mlp(h, a0, a1, W1, b1, W2, b2, gamma, beta):
    return pl.pallas_call(
        _mlp_body,
        out_shape=jax.ShapeDtypeStruct((N_NODES, FDIM), jnp.float32),
    )(h, a0, a1, W1, b1, W2, b2, gamma, beta)


def _sc_message_pass(h, src, dst, e):
    mesh = plsc.VectorSubcoreMesh(core_axis_name="c", subcore_axis_name="s")

    @functools.partial(
        pl.kernel,
        mesh=mesh,
        out_type=jax.ShapeDtypeStruct((NC, N_PAD, FDIM), jnp.float32),
        scratch_types=[
            pltpu.VMEM((CHUNK,), jnp.int32),
            pltpu.VMEM((CHUNK,), jnp.int32),
            pltpu.VMEM((CHUNK, FDIM), jnp.float32),
            pltpu.VMEM((CHUNK, FDIM), jnp.float32),
            pltpu.VMEM_SHARED((N_PAD, FDIM), jnp.float32),
            pltpu.SemaphoreType.DMA,
        ],
    )
    def mp(h_hbm, src_hbm, dst_hbm, e_hbm, out_hbm,
           sidx, didx, ebuf, hbuf, agg, sem):
        c = lax.axis_index("c")
        s = lax.axis_index("s")

        def zrow(r, carry):
            zv = jnp.zeros((16,), jnp.float32)
            for j in range(FDIM // 16):
                hbuf[r, pl.ds(j * 16, 16)] = zv
            return carry
        lax.fori_loop(0, CHUNK, zrow, 0)
        for k in range(RPT // CHUNK):
            pltpu.sync_copy(hbuf, agg.at[pl.ds(s * RPT + k * CHUNK, CHUNK)])
        plsc.subcore_barrier()

        w = c * NS + s

        def body(k, carry):
            off = w * EPW + k * CHUNK
            pltpu.sync_copy(src_hbm.at[pl.ds(off, CHUNK)], sidx)
            pltpu.sync_copy(dst_hbm.at[pl.ds(off, CHUNK)], didx)
            pltpu.sync_copy(e_hbm.at[pl.ds(off, CHUNK)], ebuf)
            pltpu.async_copy(h_hbm.at[sidx], hbuf, sem).wait()

            def row(r, rc):
                for j in range(FDIM // 16):
                    sl = pl.ds(j * 16, 16)
                    ebuf[r, sl] = jnp.maximum(ebuf[r, sl] + hbuf[r, sl], 0.0)
                return rc
            lax.fori_loop(0, CHUNK, row, 0)

            pltpu.sync_copy(ebuf, agg.at[didx], add=True)
            return carry
        lax.fori_loop(0, NCHUNKS, body, 0)

        plsc.subcore_barrier()
        for k in range(RPT // CHUNK):
            r0 = s * RPT + k * CHUNK
            pltpu.sync_copy(agg.at[pl.ds(r0, CHUNK)], out_hbm.at[c, pl.ds(r0, CHUNK)])

    return mp(h, src, dst, e)


def kernel(x, edge_index, edge_attr,
           We_0, be_0, W1_0, b1_0, W2_0, b2_0, gamma_0, beta_0,
           We_1, be_1, W1_1, b1_1, W2_1, b2_1, gamma_1, beta_1):
    pad = E_PAD - N_EDGES
    src_p = jnp.concatenate(
        [edge_index[0], jnp.arange(pad, dtype=jnp.int32) % N_NODES])
    dst_p = jnp.concatenate([edge_index[1], jnp.full((pad,), N_NODES, jnp.int32)])
    ea_p = jnp.concatenate([edge_attr, jnp.zeros((pad, EDIM), jnp.float32)])

    h = x
    for (We, be, W1, b1, W2, b2, gamma, beta) in (
        (We_0, be_0, W1_0, b1_0, W2_0, b2_0, gamma_0, beta_0),
        (We_1, be_1, W1_1, b1_1, W2_1, b2_1, gamma_1, beta_1),
    ):
        e = _tc_edge_embed(ea_p, We, be.reshape(1, FDIM))
        agg = _sc_message_pass(h, src_p, dst_p, e)
        h = _tc_mlp(h, agg[0, :N_NODES], agg[1, :N_NODES],
                    W1, b1.reshape(1, FDIM), W2, b2.reshape(1, FDIM),
                    gamma.reshape(1, FDIM), beta.reshape(1, FDIM))
    return h


# speedup vs baseline: 4.4881x; 1.6229x over previous
import functools

import jax
import jax.numpy as jnp
from jax import lax
from jax.experimental import pallas as pl
from jax.experimental.pallas import tpu as pltpu
from jax.experimental.pallas import tpu_sc as plsc

N_NODES = 10000
N_EDGES = 320000
FDIM = 128
EDIM = 16

NC = 2
NS = 16
CHUNK = 64
NBUF = 3
E_PAD = 327680
EPW = E_PAD // (NC * NS)
NCHUNKS = EPW // CHUNK
N_PAD = 10112
RPT = N_PAD // NS


def _edge_body(ea_ref, we_ref, be_ref, out_ref):
    out_ref[...] = (
        jnp.dot(ea_ref[...], we_ref[...], preferred_element_type=jnp.float32)
        + be_ref[...]
    )


def _tc_edge_embed(ea, We, be):
    BE = 4096
    return pl.pallas_call(
        _edge_body,
        grid=(E_PAD // BE,),
        in_specs=[
            pl.BlockSpec((BE, EDIM), lambda i: (i, 0)),
            pl.BlockSpec((EDIM, FDIM), lambda i: (0, 0)),
            pl.BlockSpec((1, FDIM), lambda i: (0, 0)),
        ],
        out_specs=pl.BlockSpec((BE, FDIM), lambda i: (i, 0)),
        out_shape=jax.ShapeDtypeStruct((E_PAD, FDIM), jnp.float32),
    )(ea, We, be)


def _mlp_body(h_ref, a0_ref, a1_ref, w1_ref, b1_ref, w2_ref, b2_ref,
              g_ref, bb_ref, o_ref):
    z = h_ref[...] + a0_ref[...] + a1_ref[...]
    t = jnp.dot(z, w1_ref[...], preferred_element_type=jnp.float32) + b1_ref[...]
    t = jnp.maximum(t, 0.0)
    t = jnp.dot(t, w2_ref[...], preferred_element_type=jnp.float32) + b2_ref[...]
    mu = jnp.mean(t, axis=0, keepdims=True)
    d = t - mu
    var = jnp.mean(d * d, axis=0, keepdims=True)
    o_ref[...] = jnp.maximum(
        d * lax.rsqrt(var + 1e-5) * g_ref[...] + bb_ref[...], 0.0)


def _tc_mlp(h, a0, a1, W1, b1, W2, b2, gamma, beta):
    return pl.pallas_call(
        _mlp_body,
        out_shape=jax.ShapeDtypeStruct((N_NODES, FDIM), jnp.float32),
    )(h, a0, a1, W1, b1, W2, b2, gamma, beta)


def _sc_message_pass(h, src, dst, e):
    mesh = plsc.VectorSubcoreMesh(core_axis_name="c", subcore_axis_name="s")

    @functools.partial(
        pl.kernel,
        mesh=mesh,
        out_type=jax.ShapeDtypeStruct((NC, N_PAD, FDIM), jnp.float32),
        scratch_types=[
            pltpu.VMEM((NBUF, CHUNK), jnp.int32),
            pltpu.VMEM((NBUF, CHUNK), jnp.int32),
            pltpu.VMEM((NBUF, CHUNK, FDIM), jnp.float32),
            pltpu.VMEM((NBUF, CHUNK, FDIM), jnp.float32),
            pltpu.VMEM_SHARED((N_PAD, FDIM), jnp.float32),
            pltpu.SemaphoreType.DMA((NBUF,)),
            pltpu.SemaphoreType.DMA((NBUF,)),
            pltpu.SemaphoreType.DMA((NBUF,)),
            pltpu.SemaphoreType.DMA((NBUF,)),
            pltpu.SemaphoreType.DMA((NBUF,)),
        ],
    )
    def mp(h_hbm, src_hbm, dst_hbm, e_hbm, out_hbm,
           sidx, didx, ebuf, hbuf, agg,
           sem_si, sem_di, sem_e, sem_g, sem_sc):
        c = lax.axis_index("c")
        s = lax.axis_index("s")
        base = (c * NS + s) * EPW

        def issue_in(off, b):
            pltpu.async_copy(src_hbm.at[pl.ds(off, CHUNK)], sidx.at[b],
                             sem_si.at[b])
            pltpu.async_copy(dst_hbm.at[pl.ds(off, CHUNK)], didx.at[b],
                             sem_di.at[b])
            pltpu.async_copy(e_hbm.at[pl.ds(off, CHUNK)], ebuf.at[b],
                             sem_e.at[b])

        def wait_si(b):
            pltpu.make_async_copy(src_hbm.at[pl.ds(0, CHUNK)], sidx.at[b],
                                  sem_si.at[b]).wait()

        def issue_gather(b):
            pltpu.async_copy(h_hbm.at[sidx.at[b]], hbuf.at[b], sem_g.at[b])

        def wait_eg(b):
            pltpu.make_async_copy(e_hbm.at[pl.ds(0, CHUNK)], ebuf.at[b],
                                  sem_e.at[b]).wait()
            pltpu.make_async_copy(h_hbm.at[sidx.at[b]], hbuf.at[b],
                                  sem_g.at[b]).wait()

        def compute(b):
            def row(r, rc):
                for j in range(FDIM // 16):
                    sl = pl.ds(j * 16, 16)
                    ebuf[b, r, sl] = jnp.maximum(
                        ebuf[b, r, sl] + hbuf[b, r, sl], 0.0)
                return rc
            lax.fori_loop(0, CHUNK, row, 0)

        def issue_scatter(b):
            pltpu.make_async_copy(dst_hbm.at[pl.ds(0, CHUNK)], didx.at[b],
                                  sem_di.at[b]).wait()
            pltpu.async_copy(ebuf.at[b], agg.at[didx.at[b]], sem_sc.at[b],
                             add=True)

        def wait_scatter(b):
            pltpu.make_async_copy(ebuf.at[b], agg.at[didx.at[b]],
                                  sem_sc.at[b]).wait()

        def step(off2, b, b2, first):
            wait_eg(b)
            compute(b)
            issue_scatter(b)
            if not first:
                wait_scatter(b2)
            issue_in(off2, b2)
            wait_si(b2)
            issue_gather(b2)

        def zrow(r, carry):
            zv = jnp.zeros((16,), jnp.float32)
            for j in range(FDIM // 16):
                hbuf[0, r, pl.ds(j * 16, 16)] = zv
            return carry
        lax.fori_loop(0, CHUNK, zrow, 0)
        for k in range(RPT // CHUNK):
            pltpu.sync_copy(hbuf.at[0],
                            agg.at[pl.ds(s * RPT + k * CHUNK, CHUNK)])
        rem = RPT % CHUNK
        if rem:
            pltpu.sync_copy(
                hbuf.at[0, pl.ds(0, rem)],
                agg.at[pl.ds(s * RPT + (RPT // CHUNK) * CHUNK, rem)])
        plsc.subcore_barrier()

        issue_in(base, 0)
        issue_in(base + CHUNK, 1)
        wait_si(0)
        issue_gather(0)
        wait_si(1)
        issue_gather(1)
        step(base + 2 * CHUNK, 0, 2, True)
        step(base + 3 * CHUNK, 1, 0, False)

        def body(i, carry):
            for p in range(NBUF):
                j = 2 + p
                off2 = base + (i * NBUF + j + 2) * CHUNK
                step(off2, (2 + p) % NBUF, (4 + p) % NBUF, False)
            return carry
        lax.fori_loop(0, (NCHUNKS - 4) // NBUF, body, 0)

        for b in ((NCHUNKS - 2) % NBUF, (NCHUNKS - 1) % NBUF):
            wait_eg(b)
            compute(b)
            issue_scatter(b)
        for b in range(NBUF):
            wait_scatter(b)

        plsc.subcore_barrier()
        for k in range(RPT // CHUNK):
            r0 = s * RPT + k * CHUNK
            pltpu.sync_copy(agg.at[pl.ds(r0, CHUNK)],
                            out_hbm.at[c, pl.ds(r0, CHUNK)])
        if rem:
            r0 = s * RPT + (RPT // CHUNK) * CHUNK
            pltpu.sync_copy(agg.at[pl.ds(r0, rem)],
                            out_hbm.at[c, pl.ds(r0, rem)])

    return mp(h, src, dst, e)


def kernel(x, edge_index, edge_attr,
           We_0, be_0, W1_0, b1_0, W2_0, b2_0, gamma_0, beta_0,
           We_1, be_1, W1_1, b1_1, W2_1, b2_1, gamma_1, beta_1):
    pad = E_PAD - N_EDGES
    src_p = jnp.concatenate(
        [edge_index[0], jnp.arange(pad, dtype=jnp.int32) % N_NODES])
    dst_p = jnp.concatenate([edge_index[1], jnp.full((pad,), N_NODES, jnp.int32)])
    ea_p = jnp.concatenate([edge_attr, jnp.zeros((pad, EDIM), jnp.float32)])

    h = x
    for (We, be, W1, b1, W2, b2, gamma, beta) in (
        (We_0, be_0, W1_0, b1_0, W2_0, b2_0, gamma_0, beta_0),
        (We_1, be_1, W1_1, b1_1, W2_1, b2_1, gamma_1, beta_1),
    ):
        e = _tc_edge_embed(ea_p, We, be.reshape(1, FDIM))
        agg = _sc_message_pass(h, src_p, dst_p, e)
        h = _tc_mlp(h, agg[0, :N_NODES], agg[1, :N_NODES],
                    W1, b1.reshape(1, FDIM), W2, b2.reshape(1, FDIM),
                    gamma.reshape(1, FDIM), beta.reshape(1, FDIM))
    return h


# speedup vs baseline: 4.5770x; 1.0198x over previous
import functools

import jax
import jax.numpy as jnp
from jax import lax
from jax.experimental import pallas as pl
from jax.experimental.pallas import tpu as pltpu
from jax.experimental.pallas import tpu_sc as plsc

N_NODES = 10000
N_EDGES = 320000
FDIM = 128
EDIM = 16
NGRP = FDIM // 32

NC = 2
NS = 16
CHUNK = 64
NBUF = 3
E_PAD = 327680
EPW = E_PAD // (NC * NS)
NCHUNKS = EPW // CHUNK
N_PAD = 10112
RPT = N_PAD // NS


def _edge_body(ea_ref, we_ref, be_ref, out_ref):
    out_ref[...] = (
        jnp.dot(ea_ref[...], we_ref[...], preferred_element_type=jnp.float32)
        + be_ref[...]
    )


def _tc_edge_embed(ea, We, be):
    BE = 4096
    return pl.pallas_call(
        _edge_body,
        grid=(E_PAD // BE,),
        in_specs=[
            pl.BlockSpec((BE, EDIM), lambda i: (i, 0)),
            pl.BlockSpec((EDIM, FDIM), lambda i: (0, 0)),
            pl.BlockSpec((1, FDIM), lambda i: (0, 0)),
        ],
        out_specs=pl.BlockSpec((BE, FDIM), lambda i: (i, 0)),
        out_shape=jax.ShapeDtypeStruct((E_PAD, FDIM), jnp.float32),
    )(ea, We, be)


def _mlp_body(h_ref, a_ref, w1_ref, b1_ref, w2_ref, b2_ref,
              g_ref, bb_ref, o_ref):
    z = h_ref[...] + a_ref[0] + a_ref[1]
    t = jnp.dot(z, w1_ref[...], preferred_element_type=jnp.float32) + b1_ref[...]
    t = jnp.maximum(t, 0.0)
    t = jnp.dot(t, w2_ref[...], preferred_element_type=jnp.float32) + b2_ref[...]
    mu = jnp.mean(t, axis=0, keepdims=True)
    d = t - mu
    var = jnp.mean(d * d, axis=0, keepdims=True)
    o_ref[...] = jnp.maximum(
        d * lax.rsqrt(var + 1e-5) * g_ref[...] + bb_ref[...], 0.0)


def _tc_mlp(h, agg, W1, b1, W2, b2, gamma, beta):
    return pl.pallas_call(
        _mlp_body,
        grid=(1,),
        in_specs=[
            pl.BlockSpec((N_NODES, FDIM), lambda i: (0, 0)),
            pl.BlockSpec((NC, N_NODES, FDIM), lambda i: (0, 0, 0)),
            pl.BlockSpec((FDIM, FDIM), lambda i: (0, 0)),
            pl.BlockSpec((1, FDIM), lambda i: (0, 0)),
            pl.BlockSpec((FDIM, FDIM), lambda i: (0, 0)),
            pl.BlockSpec((1, FDIM), lambda i: (0, 0)),
            pl.BlockSpec((1, FDIM), lambda i: (0, 0)),
            pl.BlockSpec((1, FDIM), lambda i: (0, 0)),
        ],
        out_specs=pl.BlockSpec((N_NODES, FDIM), lambda i: (0, 0)),
        out_shape=jax.ShapeDtypeStruct((N_NODES, FDIM), jnp.float32),
    )(h, agg, W1, b1, W2, b2, gamma, beta)


def _sc_message_pass(h, src, dst, e):
    mesh = plsc.VectorSubcoreMesh(core_axis_name="c", subcore_axis_name="s")

    @functools.partial(
        pl.kernel,
        mesh=mesh,
        out_type=jax.ShapeDtypeStruct((NC, N_PAD, FDIM), jnp.float32),
        scratch_types=[
            pltpu.VMEM((NBUF, CHUNK), jnp.int32),
            pltpu.VMEM((NBUF, CHUNK), jnp.int32),
            pltpu.VMEM((NBUF, CHUNK, FDIM), jnp.float32),
            pltpu.VMEM((NBUF, CHUNK, FDIM), jnp.float32),
            pltpu.VMEM_SHARED((N_PAD, FDIM), jnp.float32),
            pltpu.SemaphoreType.DMA((NBUF,)),
            pltpu.SemaphoreType.DMA((NBUF,)),
            pltpu.SemaphoreType.DMA((NBUF,)),
            pltpu.SemaphoreType.DMA((NBUF,)),
            pltpu.SemaphoreType.DMA((NBUF,)),
        ],
    )
    def mp(h_hbm, src_hbm, dst_hbm, e_hbm, out_hbm,
           sidx, didx, ebuf, hbuf, agg,
           sem_si, sem_di, sem_e, sem_g, sem_sc):
        c = lax.axis_index("c")
        s = lax.axis_index("s")
        base = (c * NS + s) * EPW

        def issue_in(off, b):
            pltpu.async_copy(src_hbm.at[pl.ds(off, CHUNK)], sidx.at[b],
                             sem_si.at[b])
            pltpu.async_copy(dst_hbm.at[pl.ds(off, CHUNK)], didx.at[b],
                             sem_di.at[b])
            pltpu.async_copy(e_hbm.at[pl.ds(off, CHUNK)], ebuf.at[b],
                             sem_e.at[b])

        def wait_si(b):
            pltpu.make_async_copy(src_hbm.at[pl.ds(0, CHUNK)], sidx.at[b],
                                  sem_si.at[b]).wait()

        def issue_gather(b):
            pltpu.async_copy(h_hbm.at[sidx.at[b]], hbuf.at[b], sem_g.at[b])

        def wait_eg(b):
            pltpu.make_async_copy(e_hbm.at[pl.ds(0, CHUNK)], ebuf.at[b],
                                  sem_e.at[b]).wait()
            pltpu.make_async_copy(h_hbm.at[sidx.at[b]], hbuf.at[b],
                                  sem_g.at[b]).wait()

        def compute(b):
            def row(r, rc):
                for g in range(FDIM // 16):
                    sl = pl.ds(g * 16, 16)
                    hbuf[b, r, sl] = jnp.maximum(
                        hbuf[b, r, sl] + ebuf[b, r, sl], 0.0)
                return rc
            lax.fori_loop(0, CHUNK, row, 0)

        def issue_scatter(b):
            pltpu.make_async_copy(dst_hbm.at[pl.ds(0, CHUNK)], didx.at[b],
                                  sem_di.at[b]).wait()
            pltpu.async_copy(hbuf.at[b], agg.at[didx.at[b]], sem_sc.at[b],
                             add=True)

        def wait_scatter(b):
            pltpu.make_async_copy(hbuf.at[b], agg.at[didx.at[b]],
                                  sem_sc.at[b]).wait()

        def step(off2, b, b2, first):
            wait_eg(b)
            compute(b)
            issue_scatter(b)
            if not first:
                wait_scatter(b2)
            issue_in(off2, b2)
            wait_si(b2)
            issue_gather(b2)

        def zrow(r, carry):
            zv = jnp.zeros((16,), jnp.float32)
            for j in range(FDIM // 16):
                hbuf[0, r, pl.ds(j * 16, 16)] = zv
            return carry
        lax.fori_loop(0, CHUNK, zrow, 0)
        for k in range(RPT // CHUNK):
            pltpu.sync_copy(hbuf.at[0],
                            agg.at[pl.ds(s * RPT + k * CHUNK, CHUNK)])
        rem = RPT % CHUNK
        if rem:
            pltpu.sync_copy(
                hbuf.at[0, pl.ds(0, rem)],
                agg.at[pl.ds(s * RPT + (RPT // CHUNK) * CHUNK, rem)])
        plsc.subcore_barrier()

        issue_in(base, 0)
        issue_in(base + CHUNK, 1)
        wait_si(0)
        issue_gather(0)
        wait_si(1)
        issue_gather(1)
        step(base + 2 * CHUNK, 0, 2, True)
        step(base + 3 * CHUNK, 1, 0, False)

        def body(i, carry):
            for p in range(NBUF):
                j = 2 + p
                off2 = base + (i * NBUF + j + 2) * CHUNK
                step(off2, (2 + p) % NBUF, (4 + p) % NBUF, False)
            return carry
        lax.fori_loop(0, (NCHUNKS - 4) // NBUF, body, 0)

        for b in ((NCHUNKS - 2) % NBUF, (NCHUNKS - 1) % NBUF):
            wait_eg(b)
            compute(b)
            issue_scatter(b)
        for b in range(NBUF):
            wait_scatter(b)

        plsc.subcore_barrier()
        for k in range(RPT // CHUNK):
            r0 = s * RPT + k * CHUNK
            pltpu.sync_copy(agg.at[pl.ds(r0, CHUNK)],
                            out_hbm.at[c, pl.ds(r0, CHUNK)])
        if rem:
            r0 = s * RPT + (RPT // CHUNK) * CHUNK
            pltpu.sync_copy(agg.at[pl.ds(r0, rem)],
                            out_hbm.at[c, pl.ds(r0, rem)])

    return mp(h, src, dst, e)


def kernel(x, edge_index, edge_attr,
           We_0, be_0, W1_0, b1_0, W2_0, b2_0, gamma_0, beta_0,
           We_1, be_1, W1_1, b1_1, W2_1, b2_1, gamma_1, beta_1):
    pad = E_PAD - N_EDGES
    src_p = jnp.concatenate(
        [edge_index[0], jnp.arange(pad, dtype=jnp.int32) % N_NODES])
    dst_p = jnp.concatenate([edge_index[1], jnp.full((pad,), N_NODES, jnp.int32)])
    ea_p = jnp.concatenate([edge_attr, jnp.zeros((pad, EDIM), jnp.float32)])

    h = x
    for (We, be, W1, b1, W2, b2, gamma, beta) in (
        (We_0, be_0, W1_0, b1_0, W2_0, b2_0, gamma_0, beta_0),
        (We_1, be_1, W1_1, b1_1, W2_1, b2_1, gamma_1, beta_1),
    ):
        e = _tc_edge_embed(ea_p, We, be.reshape(1, FDIM))
        agg = _sc_message_pass(h, src_p, dst_p, e)
        h = _tc_mlp(h, agg,
                    W1, b1.reshape(1, FDIM), W2, b2.reshape(1, FDIM),
                    gamma.reshape(1, FDIM), beta.reshape(1, FDIM))
    return h
